# Initial kernel scaffold; baseline (speedup 1.0000x reference)
import jax
import jax.numpy as jnp
from jax.experimental import pallas as pl


def kernel(center, normal, feature, W_l0, b_l0, g_l0, be_l0, W_f0, b_f0, g_f0, be_f0, W1, b1, g1, be1, W2, b2, g2, be2):
    raise NotImplementedError("write your pallas kernel here")


# speedup vs baseline: 5.4506x; 5.4506x over previous
import jax
import jax.numpy as jnp
from jax import lax
from jax.experimental import pallas as pl

_B, _N, _NPOINT, _NSAMPLE = 8, 4096, 512, 32
_RADIUS = 0.15
_CIN = 16
_SM = _NPOINT * _NSAMPLE
_CNT = _B * _SM


def _fps_kernel(ctT_ref, nmT_ref, ncT_ref, nnT_ref):
    ctT = ctT_ref[0]
    nmT = nmT_ref[0]
    iota_n = lax.broadcasted_iota(jnp.int32, (1, _N), 1)
    iota_p = lax.broadcasted_iota(jnp.int32, (1, _NPOINT), 1)

    def body(i, carry):
        dist, f, nc, nn = carry
        oh = (iota_n == f).astype(jnp.float32)
        c = jnp.sum(ctT * oh, axis=1, keepdims=True)
        cn = jnp.sum(nmT * oh, axis=1, keepdims=True)
        put = (iota_p == i).astype(jnp.float32)
        nc = nc + c * put
        nn = nn + cn * put
        d = jnp.sum((ctT - c) ** 2, axis=0, keepdims=True)
        dist = jnp.minimum(dist, d)
        m = jnp.max(dist)
        f_next = jnp.min(jnp.where(dist == m, iota_n, _N))
        return dist, f_next, nc, nn

    dist0 = jnp.full((1, _N), 1e10, dtype=jnp.float32)
    z = jnp.zeros((3, _NPOINT), dtype=jnp.float32)
    _, _, nc, nn = lax.fori_loop(0, _NPOINT, body, (dist0, jnp.int32(0), z, z))
    ncT_ref[0] = nc
    nnT_ref[0] = nn


def _ballq_kernel(q_ref, pT_ref, data_ref, out_ref, s_ref, ss_ref):
    q = q_ref[0]
    pT = pT_ref[0]
    data = data_ref[0]
    qq = jnp.sum(q * q, axis=1, keepdims=True)
    pp = jnp.sum(pT * pT, axis=0, keepdims=True)
    sq = qq + pp - 2.0 * jnp.dot(q, pT, preferred_element_type=jnp.float32)
    iota2 = lax.broadcasted_iota(jnp.int32, (_NPOINT, _N), 1)
    cand = jnp.where(sq <= _RADIUS * _RADIUS, iota2, _N)
    first = jnp.min(cand, axis=1, keepdims=True)
    qpad = jnp.concatenate(
        [q, jnp.zeros((_NPOINT, _CIN - 3), jnp.float32)], axis=1)
    acc_s = jnp.zeros((1, _CIN), jnp.float32)
    acc_ss = jnp.zeros((_CIN, _CIN), jnp.float32)
    for k in range(_NSAMPLE):
        idxk = jnp.min(cand, axis=1, keepdims=True)
        eff = jnp.where(idxk < _N, idxk, first)
        sel = (iota2 == eff).astype(jnp.float32)
        grp = jnp.dot(sel, data, preferred_element_type=jnp.float32)
        grp = grp - qpad
        out_ref[0, k] = grp
        acc_s = acc_s + jnp.sum(grp, axis=0, keepdims=True)
        acc_ss = acc_ss + lax.dot_general(
            grp, grp, (((0,), (0,)), ((), ())),
            preferred_element_type=jnp.float32)
        cand = jnp.where(iota2 == idxk, _N, cand)
    s_ref[0] = acc_s
    ss_ref[0] = acc_ss


def _mlp_kernel(x_ref, w_ref, c_ref, y_ref, s_ref, ss_ref):
    x = x_ref[0]
    y = jnp.maximum(
        jnp.dot(w_ref[...], x, preferred_element_type=jnp.float32)
        + c_ref[...], 0.0)
    y_ref[0] = y
    s_ref[0] = jnp.sum(y, axis=1, keepdims=True)
    ss_ref[0] = lax.dot_general(
        y, y, (((1,), (1,)), ((), ())), preferred_element_type=jnp.float32)


def _mlp_max_kernel(x_ref, w_ref, c_ref, o_ref):
    x = x_ref[0]
    y = jnp.maximum(
        jnp.dot(w_ref[...], x, preferred_element_type=jnp.float32)
        + c_ref[...], 0.0)
    m = y[:, 0:_NPOINT]
    for s in range(1, _NSAMPLE):
        m = jnp.maximum(m, y[:, s * _NPOINT:(s + 1) * _NPOINT])
    o_ref[0] = m


def _bn_affine(W, b, g, be, mu_x, cov_x):
    mu_y = W @ mu_x + b
    var_y = jnp.einsum('oi,ij,oj->o', W, cov_x, W)
    inv = g / jnp.sqrt(var_y + 1e-5)
    return W * inv[:, None], (inv * (b - mu_y) + be)[:, None]


def _moments(s, ss):
    mu = jnp.sum(s, axis=0).reshape(-1) / _CNT
    exx = jnp.sum(ss, axis=0) / _CNT
    return mu, exx - jnp.outer(mu, mu)


def _mlp_call(x, w_eff, c_eff, kern, out_shapes, out_specs):
    cout, cin = w_eff.shape
    return pl.pallas_call(
        kern,
        grid=(_B,),
        in_specs=[
            pl.BlockSpec((1, cin, _SM), lambda b: (b, 0, 0)),
            pl.BlockSpec((cout, cin), lambda b: (0, 0)),
            pl.BlockSpec((cout, 1), lambda b: (0, 0)),
        ],
        out_specs=out_specs,
        out_shape=out_shapes,
    )(x, w_eff, c_eff)


def kernel(center, normal, feature, W_l0, b_l0, g_l0, be_l0, W_f0, b_f0,
           g_f0, be_f0, W1, b1, g1, be1, W2, b2, g2, be2):
    centerT = center.transpose(0, 2, 1)
    normalT = normal.transpose(0, 2, 1)

    ncT, nnT = pl.pallas_call(
        _fps_kernel,
        grid=(_B,),
        in_specs=[
            pl.BlockSpec((1, 3, _N), lambda b: (b, 0, 0)),
            pl.BlockSpec((1, 3, _N), lambda b: (b, 0, 0)),
        ],
        out_specs=[
            pl.BlockSpec((1, 3, _NPOINT), lambda b: (b, 0, 0)),
            pl.BlockSpec((1, 3, _NPOINT), lambda b: (b, 0, 0)),
        ],
        out_shape=[
            jax.ShapeDtypeStruct((_B, 3, _NPOINT), jnp.float32),
            jax.ShapeDtypeStruct((_B, 3, _NPOINT), jnp.float32),
        ],
    )(centerT, normalT)
    new_center = ncT.transpose(0, 2, 1)
    new_normal = nnT.transpose(0, 2, 1)

    data = jnp.concatenate(
        [center, normal, feature.transpose(0, 2, 1)], axis=-1)

    grp, s1, ss1 = pl.pallas_call(
        _ballq_kernel,
        grid=(_B,),
        in_specs=[
            pl.BlockSpec((1, _NPOINT, 3), lambda b: (b, 0, 0)),
            pl.BlockSpec((1, 3, _N), lambda b: (b, 0, 0)),
            pl.BlockSpec((1, _N, _CIN), lambda b: (b, 0, 0)),
        ],
        out_specs=[
            pl.BlockSpec((1, _NSAMPLE, _NPOINT, _CIN),
                         lambda b: (b, 0, 0, 0)),
            pl.BlockSpec((1, 1, _CIN), lambda b: (b, 0, 0)),
            pl.BlockSpec((1, _CIN, _CIN), lambda b: (b, 0, 0)),
        ],
        out_shape=[
            jax.ShapeDtypeStruct((_B, _NSAMPLE, _NPOINT, _CIN), jnp.float32),
            jax.ShapeDtypeStruct((_B, 1, _CIN), jnp.float32),
            jax.ShapeDtypeStruct((_B, _CIN, _CIN), jnp.float32),
        ],
    )(new_center, centerT, data)

    x1 = grp.transpose(0, 3, 1, 2).reshape(_B, _CIN, _SM)

    mu1, cov1 = _moments(s1, ss1)
    Wl_eff, cl = _bn_affine(W_l0, b_l0, g_l0, be_l0, mu1[:3], cov1[:3, :3])
    Wf_eff, cf = _bn_affine(W_f0, b_f0, g_f0, be_f0, mu1[3:], cov1[3:, 3:])
    W0_eff = jnp.concatenate([Wl_eff, Wf_eff], axis=1)
    c0 = cl + cf

    def stat_out(cout):
        return ([
            pl.BlockSpec((1, cout, _SM), lambda b: (b, 0, 0)),
            pl.BlockSpec((1, cout, 1), lambda b: (b, 0, 0)),
            pl.BlockSpec((1, cout, cout), lambda b: (b, 0, 0)),
        ], [
            jax.ShapeDtypeStruct((_B, cout, _SM), jnp.float32),
            jax.ShapeDtypeStruct((_B, cout, 1), jnp.float32),
            jax.ShapeDtypeStruct((_B, cout, cout), jnp.float32),
        ])

    specs64, shapes64 = stat_out(64)
    x2, s2, ss2 = _mlp_call(x1, W0_eff, c0, _mlp_kernel, shapes64, specs64)
    mu2, cov2 = _moments(s2.reshape(_B, 1, 64), ss2)
    W1_eff, c1 = _bn_affine(W1, b1, g1, be1, mu2, cov2)

    x3, s3, ss3 = _mlp_call(x2, W1_eff, c1, _mlp_kernel, shapes64, specs64)
    mu3, cov3 = _moments(s3.reshape(_B, 1, 64), ss3)
    W2_eff, c2 = _bn_affine(W2, b2, g2, be2, mu3, cov3)

    out = _mlp_call(
        x3, W2_eff, c2, _mlp_max_kernel,
        jax.ShapeDtypeStruct((_B, 128, _NPOINT), jnp.float32),
        pl.BlockSpec((1, 128, _NPOINT), lambda b: (b, 0, 0)))

    return new_center, new_normal, out


# speedup vs baseline: 13.1227x; 2.4076x over previous
import jax
import jax.numpy as jnp
from jax import lax
from jax.experimental import pallas as pl

_B, _N, _NPOINT, _NSAMPLE = 8, 4096, 512, 32
_RADIUS = 0.15
_CIN = 16
_SM = _NPOINT * _NSAMPLE
_CNT = _B * _SM


def _fps_kernel(ct_ref, nm_ref, nc_ref, nn_ref):
    ct = ct_ref[...]
    nm = nm_ref[...]
    iota_n = lax.broadcasted_iota(jnp.int32, (_B, _N), 1)
    iota_p = lax.broadcasted_iota(jnp.int32, (1, 1, _NPOINT), 2)

    def body(i, carry):
        dist, f, nc, nn = carry
        oh = (iota_n == f).astype(jnp.float32)
        c = jnp.sum(ct * oh[None], axis=2, keepdims=True)
        cn = jnp.sum(nm * oh[None], axis=2, keepdims=True)
        put = (iota_p == i).astype(jnp.float32)
        nc = nc + c * put
        nn = nn + cn * put
        d = jnp.sum((ct - c) ** 2, axis=0)
        dist = jnp.minimum(dist, d)
        m = jnp.max(dist, axis=1, keepdims=True)
        f_next = jnp.min(jnp.where(dist == m, iota_n, _N), axis=1,
                         keepdims=True)
        return dist, f_next, nc, nn

    dist0 = jnp.full((_B, _N), 1e10, dtype=jnp.float32)
    f0 = jnp.zeros((_B, 1), dtype=jnp.int32)
    z = jnp.zeros((3, _B, _NPOINT), dtype=jnp.float32)
    _, _, nc, nn = lax.fori_loop(0, _NPOINT, body, (dist0, f0, z, z))
    nc_ref[...] = nc
    nn_ref[...] = nn


def _ballq_kernel(q_ref, pT_ref, data_ref, out_ref, s_ref, ss_ref):
    q = q_ref[0]
    pT = pT_ref[0]
    data = data_ref[0]
    qq = jnp.sum(q * q, axis=1, keepdims=True)
    pp = jnp.sum(pT * pT, axis=0, keepdims=True)
    sq = qq + pp - 2.0 * jnp.dot(q, pT, preferred_element_type=jnp.float32)
    iota2 = lax.broadcasted_iota(jnp.int32, (_NPOINT, _N), 1)
    cand = jnp.where(sq <= _RADIUS * _RADIUS, iota2, _N)
    first = jnp.min(cand, axis=1, keepdims=True)
    qpad = jnp.concatenate(
        [q, jnp.zeros((_NPOINT, _CIN - 3), jnp.float32)], axis=1)
    acc_s = jnp.zeros((1, _CIN), jnp.float32)
    acc_ss = jnp.zeros((_CIN, _CIN), jnp.float32)
    for k in range(_NSAMPLE):
        idxk = jnp.min(cand, axis=1, keepdims=True)
        eff = jnp.where(idxk < _N, idxk, first)
        sel = (iota2 == eff).astype(jnp.float32)
        grp = jnp.dot(sel, data, preferred_element_type=jnp.float32)
        grp = grp - qpad
        out_ref[0, k] = grp
        acc_s = acc_s + jnp.sum(grp, axis=0, keepdims=True)
        acc_ss = acc_ss + lax.dot_general(
            grp, grp, (((0,), (0,)), ((), ())),
            preferred_element_type=jnp.float32)
        cand = jnp.where(iota2 == idxk, _N, cand)
    s_ref[0] = acc_s
    ss_ref[0] = acc_ss


def _mlp_kernel(x_ref, w_ref, c_ref, y_ref, s_ref, ss_ref):
    x = x_ref[0]
    y = jnp.maximum(
        jnp.dot(w_ref[...], x, preferred_element_type=jnp.float32)
        + c_ref[...], 0.0)
    y_ref[0] = y
    s_ref[0] = jnp.sum(y, axis=1, keepdims=True)
    ss_ref[0] = lax.dot_general(
        y, y, (((1,), (1,)), ((), ())), preferred_element_type=jnp.float32)


def _mlp_max_kernel(x_ref, w_ref, c_ref, o_ref):
    x = x_ref[0]
    y = jnp.maximum(
        jnp.dot(w_ref[...], x, preferred_element_type=jnp.float32)
        + c_ref[...], 0.0)
    m = y[:, 0:_NPOINT]
    for s in range(1, _NSAMPLE):
        m = jnp.maximum(m, y[:, s * _NPOINT:(s + 1) * _NPOINT])
    o_ref[0] = m


def _bn_affine(W, b, g, be, mu_x, cov_x):
    mu_y = W @ mu_x + b
    var_y = jnp.einsum('oi,ij,oj->o', W, cov_x, W)
    inv = g / jnp.sqrt(var_y + 1e-5)
    return W * inv[:, None], (inv * (b - mu_y) + be)[:, None]


def _moments(s, ss):
    mu = jnp.sum(s, axis=0).reshape(-1) / _CNT
    exx = jnp.sum(ss, axis=0) / _CNT
    return mu, exx - jnp.outer(mu, mu)


def _mlp_call(x, w_eff, c_eff, kern, out_shapes, out_specs):
    cout, cin = w_eff.shape
    return pl.pallas_call(
        kern,
        grid=(_B,),
        in_specs=[
            pl.BlockSpec((1, cin, _SM), lambda b: (b, 0, 0)),
            pl.BlockSpec((cout, cin), lambda b: (0, 0)),
            pl.BlockSpec((cout, 1), lambda b: (0, 0)),
        ],
        out_specs=out_specs,
        out_shape=out_shapes,
    )(x, w_eff, c_eff)


def kernel(center, normal, feature, W_l0, b_l0, g_l0, be_l0, W_f0, b_f0,
           g_f0, be_f0, W1, b1, g1, be1, W2, b2, g2, be2):
    centerT = center.transpose(0, 2, 1)
    normalT = normal.transpose(0, 2, 1)

    ncS, nnS = pl.pallas_call(
        _fps_kernel,
        out_shape=[
            jax.ShapeDtypeStruct((3, _B, _NPOINT), jnp.float32),
            jax.ShapeDtypeStruct((3, _B, _NPOINT), jnp.float32),
        ],
    )(center.transpose(2, 0, 1), normal.transpose(2, 0, 1))
    new_center = ncS.transpose(1, 2, 0)
    new_normal = nnS.transpose(1, 2, 0)

    data = jnp.concatenate(
        [center, normal, feature.transpose(0, 2, 1)], axis=-1)

    grp, s1, ss1 = pl.pallas_call(
        _ballq_kernel,
        grid=(_B,),
        in_specs=[
            pl.BlockSpec((1, _NPOINT, 3), lambda b: (b, 0, 0)),
            pl.BlockSpec((1, 3, _N), lambda b: (b, 0, 0)),
            pl.BlockSpec((1, _N, _CIN), lambda b: (b, 0, 0)),
        ],
        out_specs=[
            pl.BlockSpec((1, _NSAMPLE, _NPOINT, _CIN),
                         lambda b: (b, 0, 0, 0)),
            pl.BlockSpec((1, 1, _CIN), lambda b: (b, 0, 0)),
            pl.BlockSpec((1, _CIN, _CIN), lambda b: (b, 0, 0)),
        ],
        out_shape=[
            jax.ShapeDtypeStruct((_B, _NSAMPLE, _NPOINT, _CIN), jnp.float32),
            jax.ShapeDtypeStruct((_B, 1, _CIN), jnp.float32),
            jax.ShapeDtypeStruct((_B, _CIN, _CIN), jnp.float32),
        ],
    )(new_center, centerT, data)

    x1 = grp.transpose(0, 3, 1, 2).reshape(_B, _CIN, _SM)

    mu1, cov1 = _moments(s1, ss1)
    Wl_eff, cl = _bn_affine(W_l0, b_l0, g_l0, be_l0, mu1[:3], cov1[:3, :3])
    Wf_eff, cf = _bn_affine(W_f0, b_f0, g_f0, be_f0, mu1[3:], cov1[3:, 3:])
    W0_eff = jnp.concatenate([Wl_eff, Wf_eff], axis=1)
    c0 = cl + cf

    def stat_out(cout):
        return ([
            pl.BlockSpec((1, cout, _SM), lambda b: (b, 0, 0)),
            pl.BlockSpec((1, cout, 1), lambda b: (b, 0, 0)),
            pl.BlockSpec((1, cout, cout), lambda b: (b, 0, 0)),
        ], [
            jax.ShapeDtypeStruct((_B, cout, _SM), jnp.float32),
            jax.ShapeDtypeStruct((_B, cout, 1), jnp.float32),
            jax.ShapeDtypeStruct((_B, cout, cout), jnp.float32),
        ])

    specs64, shapes64 = stat_out(64)
    x2, s2, ss2 = _mlp_call(x1, W0_eff, c0, _mlp_kernel, shapes64, specs64)
    mu2, cov2 = _moments(s2.reshape(_B, 1, 64), ss2)
    W1_eff, c1 = _bn_affine(W1, b1, g1, be1, mu2, cov2)

    x3, s3, ss3 = _mlp_call(x2, W1_eff, c1, _mlp_kernel, shapes64, specs64)
    mu3, cov3 = _moments(s3.reshape(_B, 1, 64), ss3)
    W2_eff, c2 = _bn_affine(W2, b2, g2, be2, mu3, cov3)

    out = _mlp_call(
        x3, W2_eff, c2, _mlp_max_kernel,
        jax.ShapeDtypeStruct((_B, 128, _NPOINT), jnp.float32),
        pl.BlockSpec((1, 128, _NPOINT), lambda b: (b, 0, 0)))

    return new_center, new_normal, out
